# baseline (speedup 1.0000x reference)
import functools

import jax
import jax.numpy as jnp
from jax import lax
from jax.experimental import pallas as pl
from jax.experimental.pallas import tpu as pltpu
from jax.experimental.pallas import tpu_sc as plsc

N = 10000
E = 80000
P = 2
B = 16
HID = 64
GDIM = 50

NPAD = 10240
TILES = 16
NCHUNK = E // 128
KSUB = 40
LAST = NCHUNK - 15 * KSUB
ROWS_PT = NPAD // TILES
BLK = 2048


def _conv(x, w, b, stride=(1, 1), padding=((0, 0), (0, 0))):
    y = lax.conv_general_dilated(x, w, window_strides=stride, padding=padding,
                                 dimension_numbers=('NCHW', 'OIHW', 'NCHW'))
    return y + b[None, :, None, None]


def _stack(t, c1w, c1b, c2w, c2b, c3w, c3b, lw, lb):
    h = _conv(t, c1w, c1b, padding=((P, P), (0, 0)))
    h = _conv(h, c2w, c2b)
    h = _conv(h, c3w, c3b, stride=(1, B // 16))
    return h.reshape(h.shape[0], -1) @ lw.T + lb


def _affine_of_stack(cin, params):
    dim = cin * P * B
    basis = jnp.concatenate(
        [jnp.eye(dim, dtype=jnp.float32), jnp.zeros((1, dim), jnp.float32)], 0
    ).reshape(dim + 1, cin, P, B)
    out = _stack(basis, *params)
    return out[:dim] - out[dim][None], out[dim]


def _sc_body(x32, epf, eaf, z32, z1, oxx, osc,
             jv, iv, eav, rows, acc_ea, acc_deg, red_in, red_out,
             acc_x, stage_s, sem, sem2, sem3):
    c = lax.axis_index("c")
    s = lax.axis_index("s")
    r0 = s * ROWS_PT
    pltpu.sync_copy(z32, acc_x.at[pl.ds(r0, ROWS_PT)])
    pltpu.sync_copy(z1, acc_ea)
    pltpu.sync_copy(z1, acc_deg)
    nb = jnp.where(s < 12, 10, 9)
    base = jnp.where(s < 12, s * 5120, 61440 + (s - 12) * 4608)

    def stg(i, carry):
        pltpu.async_copy(epf.at[c, pl.ds(base + i * 512, 512)], jv.at[i], sem3)
        pltpu.async_copy(epf.at[1 - c, pl.ds(base + i * 512, 512)], iv.at[i],
                         sem3)
        pltpu.async_copy(eaf.at[pl.ds(base + i * 512, 512)], eav.at[i], sem3)
        return carry

    lax.fori_loop(0, nb, stg, 0)

    @pl.when(s == 15)
    def _():
        pltpu.async_copy(epf.at[c, pl.ds(79872, 128)],
                         jv.at[9, pl.ds(0, 128)], sem3)
        pltpu.async_copy(epf.at[1 - c, pl.ds(79872, 128)],
                         iv.at[9, pl.ds(0, 128)], sem3)
        pltpu.async_copy(eaf.at[pl.ds(79872, 128)],
                         eav.at[9, pl.ds(0, 128)], sem3)

    def stg_drain(i, carry):
        pltpu.make_async_copy(epf.at[c, pl.ds(base + i * 512, 512)], jv.at[i],
                              sem3).wait()
        pltpu.make_async_copy(epf.at[1 - c, pl.ds(base + i * 512, 512)],
                              iv.at[i], sem3).wait()
        pltpu.make_async_copy(eaf.at[pl.ds(base + i * 512, 512)], eav.at[i],
                              sem3).wait()
        return carry

    lax.fori_loop(0, nb, stg_drain, 0)

    @pl.when(s == 15)
    def _():
        pltpu.make_async_copy(epf.at[c, pl.ds(79872, 128)],
                              jv.at[9, pl.ds(0, 128)], sem3).wait()
        pltpu.make_async_copy(epf.at[1 - c, pl.ds(79872, 128)],
                              iv.at[9, pl.ds(0, 128)], sem3).wait()
        pltpu.make_async_copy(eaf.at[pl.ds(79872, 128)],
                              eav.at[9, pl.ds(0, 128)], sem3).wait()

    plsc.subcore_barrier()

    for b in range(2):
        pltpu.async_copy(x32.at[jv.at[b]], rows.at[b], sem)

    ones16 = jnp.ones((16,), jnp.float32)

    def step(k, carry):
        b = lax.rem(k, 4)
        pltpu.make_async_copy(x32.at[jv.at[k]], rows.at[b], sem).wait()
        pltpu.async_copy(rows.at[b], acc_x.at[iv.at[k]], sem2, add=True)
        for m in range(32):
            idxv = iv[k, pl.ds(m * 16, 16)]
            vals = eav[k, pl.ds(m * 16, 16)]
            plsc.addupdate_scatter(acc_ea, [idxv], vals)
            plsc.addupdate_scatter(acc_deg, [idxv], ones16)

        @pl.when(k >= 2)
        def _():
            b2 = lax.rem(k - 2, 4)
            pltpu.make_async_copy(rows.at[b2], acc_x.at[iv.at[k - 2]],
                                  sem2).wait()

        @pl.when(k + 2 < nb)
        def _():
            pltpu.async_copy(x32.at[jv.at[k + 2]],
                             rows.at[lax.rem(k + 2, 4)], sem)

        return carry

    lax.fori_loop(0, nb, step, 0)
    pltpu.make_async_copy(rows.at[0], acc_x.at[iv.at[0]], sem2).wait()
    pltpu.make_async_copy(rows.at[0], acc_x.at[iv.at[0]], sem2).wait()

    @pl.when(s == 15)
    def _():
        pltpu.async_copy(x32.at[jv.at[9, pl.ds(0, 128)]],
                         rows.at[0, pl.ds(0, 128)], sem).wait()
        pltpu.sync_copy(rows.at[0, pl.ds(0, 128)],
                        acc_x.at[iv.at[9, pl.ds(0, 128)]], add=True)
        for m in range(8):
            idxv = iv[9, pl.ds(m * 16, 16)]
            vals = eav[9, pl.ds(m * 16, 16)]
            plsc.addupdate_scatter(acc_ea, [idxv], vals)
            plsc.addupdate_scatter(acc_deg, [idxv], ones16)

    pltpu.sync_copy(acc_ea, stage_s.at[s, 0])
    pltpu.sync_copy(acc_deg, stage_s.at[s, 1])
    plsc.subcore_barrier()
    pltpu.sync_copy(stage_s.at[:, :, pl.ds(r0, ROWS_PT)], red_in)

    def red(g, carry):
        for r in range(2):
            acc = jnp.zeros((16,), jnp.float32)
            for t in range(TILES):
                acc = acc + red_in[t, r, pl.ds(g * 16, 16)]
            red_out[r, pl.ds(g * 16, 16)] = acc
        return carry

    lax.fori_loop(0, ROWS_PT // 16, red, 0)
    pltpu.sync_copy(red_out, osc.at[c, :, pl.ds(r0, ROWS_PT)])

    pltpu.sync_copy(acc_x.at[pl.ds(r0, ROWS_PT)],
                    oxx.at[c, pl.ds(r0, ROWS_PT)])


@functools.lru_cache(maxsize=1)
def _sc_segment():
    return pl.kernel(
        _sc_body,
        out_type=[
            jax.ShapeDtypeStruct((2, NPAD, 32), jnp.bfloat16),
            jax.ShapeDtypeStruct((2, 2, NPAD), jnp.float32),
        ],
        mesh=plsc.VectorSubcoreMesh(core_axis_name="c", subcore_axis_name="s"),
        compiler_params=pltpu.CompilerParams(use_tc_tiling_on_sc=False,
                                             needs_layout_passes=False),
        scratch_types=[
            pltpu.VMEM((10, 512), jnp.int32),
            pltpu.VMEM((10, 512), jnp.int32),
            pltpu.VMEM((10, 512), jnp.float32),
            pltpu.VMEM((4, 512, 32), jnp.bfloat16),
            pltpu.VMEM((NPAD,), jnp.float32),
            pltpu.VMEM((NPAD,), jnp.float32),
            pltpu.VMEM((TILES, 2, ROWS_PT), jnp.float32),
            pltpu.VMEM((2, ROWS_PT), jnp.float32),
            pltpu.VMEM_SHARED((NPAD, 32), jnp.bfloat16),
            pltpu.VMEM_SHARED((TILES, 2, NPAD), jnp.float32),
            pltpu.SemaphoreType.DMA,
            pltpu.SemaphoreType.DMA,
            pltpu.SemaphoreType.DMA,
        ],
    )


def _tc_body(xs, fxr, bxr, osc, W, Ws, bg, GM, gmb, FM, fmb, out_ref):
    pid = pl.program_id(0)
    x = xs[...].astype(jnp.float32)
    eaf = osc[0, 0, :]
    degf = osc[0, 1, :]
    eab = osc[1, 0, :]
    degb = osc[1, 1, :]
    Wv = W[...]
    S = jnp.stack([eaf, degf, eab, degb], axis=1)
    h = (jnp.dot(x * degf[:, None], Wv[0:32], preferred_element_type=jnp.float32)
         + jnp.dot(x * degb[:, None], Wv[32:64], preferred_element_type=jnp.float32)
         + jnp.dot(fxr[0].astype(jnp.float32), Wv[64:96], preferred_element_type=jnp.float32)
         + jnp.dot(bxr[0].astype(jnp.float32), Wv[96:128], preferred_element_type=jnp.float32)
         + jnp.dot(S, Ws[...], preferred_element_type=jnp.float32) + bg[...])
    g = jax.nn.sigmoid(jnp.dot(h, GM[...], preferred_element_type=jnp.float32)
                       + gmb[...])
    hv = jnp.dot(h, FM[...], preferred_element_type=jnp.float32) + fmb[...]
    rid = pid * BLK + lax.broadcasted_iota(jnp.int32, (BLK, 1), 0)
    part = jnp.sum(jnp.where(rid < N, g * hv, 0.0), axis=0, keepdims=True)

    @pl.when(pid == 0)
    def _():
        out_ref[...] = jnp.zeros_like(out_ref)

    out_ref[...] += part


def _tc_reduce(xs, oxx, osc, W, Ws, bg, GM, gmb, FM, fmb):
    return pl.pallas_call(
        _tc_body,
        grid=(NPAD // BLK,),
        in_specs=[
            pl.BlockSpec((BLK, 32), lambda i: (i, 0)),
            pl.BlockSpec((1, BLK, 32), lambda i: (0, i, 0)),
            pl.BlockSpec((1, BLK, 32), lambda i: (1, i, 0)),
            pl.BlockSpec((2, 2, BLK), lambda i: (0, 0, i)),
            pl.BlockSpec((128, 64), lambda i: (0, 0)),
            pl.BlockSpec((4, 64), lambda i: (0, 0)),
            pl.BlockSpec((1, 64), lambda i: (0, 0)),
            pl.BlockSpec((64, 64), lambda i: (0, 0)),
            pl.BlockSpec((1, 64), lambda i: (0, 0)),
            pl.BlockSpec((64, 64), lambda i: (0, 0)),
            pl.BlockSpec((1, 64), lambda i: (0, 0)),
        ],
        out_specs=pl.BlockSpec((1, 64), lambda i: (0, 0)),
        out_shape=jax.ShapeDtypeStruct((1, 64), jnp.float32),
    )(xs, oxx, oxx, osc, W, Ws, bg, GM, gmb, FM, fmb)


def _direction_mats(stack_params, nd_w, nd_b, Ag):
    A, b = _affine_of_stack(2, stack_params)
    Wnd = nd_w.reshape(P * B, HID + 1).T
    M = A @ Wnd[:HID]
    w_ea = Wnd[HID]
    cst = b @ Wnd[:HID] + nd_b.reshape(P * B)
    return M[:32] @ Ag, M[32:] @ Ag, w_ea @ Ag, cst @ Ag


def kernel(x, edge_attr, edge_index,
           fwd_c1_w, fwd_c1_b, fwd_c2_w, fwd_c2_b, fwd_c3_w, fwd_c3_b,
           fwd_lin_w, fwd_lin_b, fwd_nd_w, fwd_nd_b,
           bwd_c1_w, bwd_c1_b, bwd_c2_w, bwd_c2_b, bwd_c3_w, bwd_c3_b,
           bwd_lin_w, bwd_lin_b, bwd_nd_w, bwd_nd_b,
           g_c1_w, g_c1_b, g_c2_w, g_c2_b, g_c3_w, g_c3_b,
           g_lin_w, g_lin_b, gm_w, gm_b, fm_w, fm_b):
    f32 = jnp.float32
    Ag, bg = _affine_of_stack(1, (g_c1_w, g_c1_b, g_c2_w, g_c2_b,
                                  g_c3_w, g_c3_b, g_lin_w, g_lin_b))
    Mfi, Mfj, wf, cf = _direction_mats(
        (fwd_c1_w, fwd_c1_b, fwd_c2_w, fwd_c2_b, fwd_c3_w, fwd_c3_b,
         fwd_lin_w, fwd_lin_b), fwd_nd_w, fwd_nd_b, Ag)
    Mbi, Mbj, wb, cb = _direction_mats(
        (bwd_c1_w, bwd_c1_b, bwd_c2_w, bwd_c2_b, bwd_c3_w, bwd_c3_b,
         bwd_lin_w, bwd_lin_b), bwd_nd_w, bwd_nd_b, Ag)
    W_all = jnp.concatenate([Mfi, Mbi, Mfj, Mbj], axis=0)
    Ws = jnp.stack([wf, cf, wb, cb], axis=0)
    GM = jnp.zeros((HID, 64), f32).at[:, :GDIM].set(gm_w.T)
    gmb = jnp.zeros((1, 64), f32).at[0, :GDIM].set(gm_b)
    FM = jnp.zeros((HID, 64), f32).at[:, :GDIM].set(fm_w.T)
    fmb = jnp.zeros((1, 64), f32).at[0, :GDIM].set(fm_b)

    x32p = jnp.pad(x.reshape(N, P * B), ((0, NPAD - N), (0, 0))
                   ).astype(jnp.bfloat16)
    z32 = jnp.zeros((ROWS_PT, 32), jnp.bfloat16)
    z1 = jnp.zeros((NPAD,), f32)

    oxx, osc = _sc_segment()(x32p, edge_index, edge_attr, z32, z1)

    out = _tc_reduce(x32p, oxx, osc, W_all, Ws,
                     bg.reshape(1, HID), GM, gmb, FM, fmb)
    return out[0, :GDIM]


# speedup vs baseline: 1.0271x; 1.0271x over previous
import functools

import jax
import jax.numpy as jnp
from jax import lax
from jax.experimental import pallas as pl
from jax.experimental.pallas import tpu as pltpu
from jax.experimental.pallas import tpu_sc as plsc

N = 10000
E = 80000
P = 2
B = 16
HID = 64
GDIM = 50

NPAD = 10240
TILES = 16
NCHUNK = E // 128
KSUB = 40
LAST = NCHUNK - 15 * KSUB
ROWS_PT = NPAD // TILES
BLK = 2048


def _conv(x, w, b, stride=(1, 1), padding=((0, 0), (0, 0))):
    y = lax.conv_general_dilated(x, w, window_strides=stride, padding=padding,
                                 dimension_numbers=('NCHW', 'OIHW', 'NCHW'))
    return y + b[None, :, None, None]


def _stack(t, c1w, c1b, c2w, c2b, c3w, c3b, lw, lb):
    h = _conv(t, c1w, c1b, padding=((P, P), (0, 0)))
    h = _conv(h, c2w, c2b)
    h = _conv(h, c3w, c3b, stride=(1, B // 16))
    return h.reshape(h.shape[0], -1) @ lw.T + lb


def _affine_of_stack(cin, params):
    dim = cin * P * B
    basis = jnp.concatenate(
        [jnp.eye(dim, dtype=jnp.float32), jnp.zeros((1, dim), jnp.float32)], 0
    ).reshape(dim + 1, cin, P, B)
    out = _stack(basis, *params)
    return out[:dim] - out[dim][None], out[dim]


def _sc_body(x32, epf, eaf, z32, z1, oxx, osc,
             jv, iv, eav, rows, acc_ea, acc_deg, red_in, red_out,
             acc_x, stage_s, sem, sem2, sem3):
    c = lax.axis_index("c")
    s = lax.axis_index("s")
    r0 = s * ROWS_PT
    pltpu.sync_copy(z32, acc_x.at[pl.ds(r0, ROWS_PT)])
    pltpu.sync_copy(z1, acc_ea)
    pltpu.sync_copy(z1, acc_deg)
    nb = jnp.where(s < 14, 5, 4)
    base = jnp.where(s < 14, s * 5120, 71680 + (s - 14) * 4096)

    def stg(i, carry):
        pltpu.async_copy(epf.at[c, pl.ds(base + i * 1024, 1024)], jv.at[i], sem3)
        pltpu.async_copy(epf.at[1 - c, pl.ds(base + i * 1024, 1024)], iv.at[i],
                         sem3)
        pltpu.async_copy(eaf.at[pl.ds(base + i * 1024, 1024)], eav.at[i], sem3)
        return carry

    lax.fori_loop(0, nb, stg, 0)

    @pl.when(s == 15)
    def _():
        pltpu.async_copy(epf.at[c, pl.ds(79872, 128)],
                         jv.at[4, pl.ds(0, 128)], sem3)
        pltpu.async_copy(epf.at[1 - c, pl.ds(79872, 128)],
                         iv.at[4, pl.ds(0, 128)], sem3)
        pltpu.async_copy(eaf.at[pl.ds(79872, 128)],
                         eav.at[4, pl.ds(0, 128)], sem3)

    def stg_drain(i, carry):
        pltpu.make_async_copy(epf.at[c, pl.ds(base + i * 1024, 1024)], jv.at[i],
                              sem3).wait()
        pltpu.make_async_copy(epf.at[1 - c, pl.ds(base + i * 1024, 1024)],
                              iv.at[i], sem3).wait()
        pltpu.make_async_copy(eaf.at[pl.ds(base + i * 1024, 1024)], eav.at[i],
                              sem3).wait()
        return carry

    lax.fori_loop(0, nb, stg_drain, 0)

    @pl.when(s == 15)
    def _():
        pltpu.make_async_copy(epf.at[c, pl.ds(79872, 128)],
                              jv.at[4, pl.ds(0, 128)], sem3).wait()
        pltpu.make_async_copy(epf.at[1 - c, pl.ds(79872, 128)],
                              iv.at[4, pl.ds(0, 128)], sem3).wait()
        pltpu.make_async_copy(eaf.at[pl.ds(79872, 128)],
                              eav.at[4, pl.ds(0, 128)], sem3).wait()

    plsc.subcore_barrier()

    for b in range(2):
        pltpu.async_copy(x32.at[jv.at[b]], rows.at[b], sem)

    ones16 = jnp.ones((16,), jnp.float32)

    def step(k, carry):
        b = lax.rem(k, 2)
        pltpu.make_async_copy(x32.at[jv.at[k]], rows.at[b], sem).wait()
        pltpu.async_copy(rows.at[b], acc_x.at[iv.at[k]], sem2, add=True)
        for m in range(64):
            idxv = iv[k, pl.ds(m * 16, 16)]
            vals = eav[k, pl.ds(m * 16, 16)]
            plsc.addupdate_scatter(acc_ea, [idxv], vals)
            plsc.addupdate_scatter(acc_deg, [idxv], ones16)
        pltpu.make_async_copy(rows.at[b], acc_x.at[iv.at[k]], sem2).wait()

        @pl.when(k + 2 < nb)
        def _():
            pltpu.async_copy(x32.at[jv.at[k + 2]],
                             rows.at[lax.rem(k + 2, 2)], sem)

        return carry

    lax.fori_loop(0, nb, step, 0)

    @pl.when(s == 15)
    def _():
        pltpu.async_copy(x32.at[jv.at[4, pl.ds(0, 128)]],
                         rows.at[0, pl.ds(0, 128)], sem).wait()
        pltpu.sync_copy(rows.at[0, pl.ds(0, 128)],
                        acc_x.at[iv.at[4, pl.ds(0, 128)]], add=True)
        for m in range(8):
            idxv = iv[4, pl.ds(m * 16, 16)]
            vals = eav[4, pl.ds(m * 16, 16)]
            plsc.addupdate_scatter(acc_ea, [idxv], vals)
            plsc.addupdate_scatter(acc_deg, [idxv], ones16)

    pltpu.sync_copy(acc_ea, stage_s.at[s, 0])
    pltpu.sync_copy(acc_deg, stage_s.at[s, 1])
    plsc.subcore_barrier()
    pltpu.sync_copy(stage_s.at[:, :, pl.ds(r0, ROWS_PT)], red_in)

    def red(g, carry):
        for r in range(2):
            acc = jnp.zeros((16,), jnp.float32)
            for t in range(TILES):
                acc = acc + red_in[t, r, pl.ds(g * 16, 16)]
            red_out[r, pl.ds(g * 16, 16)] = acc
        return carry

    lax.fori_loop(0, ROWS_PT // 16, red, 0)
    pltpu.sync_copy(red_out, osc.at[c, :, pl.ds(r0, ROWS_PT)])

    pltpu.sync_copy(acc_x.at[pl.ds(r0, ROWS_PT)],
                    oxx.at[c, pl.ds(r0, ROWS_PT)])


@functools.lru_cache(maxsize=1)
def _sc_segment():
    return pl.kernel(
        _sc_body,
        out_type=[
            jax.ShapeDtypeStruct((2, NPAD, 32), jnp.bfloat16),
            jax.ShapeDtypeStruct((2, 2, NPAD), jnp.float32),
        ],
        mesh=plsc.VectorSubcoreMesh(core_axis_name="c", subcore_axis_name="s"),
        compiler_params=pltpu.CompilerParams(use_tc_tiling_on_sc=False,
                                             needs_layout_passes=False),
        scratch_types=[
            pltpu.VMEM((5, 1024), jnp.int32),
            pltpu.VMEM((5, 1024), jnp.int32),
            pltpu.VMEM((5, 1024), jnp.float32),
            pltpu.VMEM((2, 1024, 32), jnp.bfloat16),
            pltpu.VMEM((NPAD,), jnp.float32),
            pltpu.VMEM((NPAD,), jnp.float32),
            pltpu.VMEM((TILES, 2, ROWS_PT), jnp.float32),
            pltpu.VMEM((2, ROWS_PT), jnp.float32),
            pltpu.VMEM_SHARED((NPAD, 32), jnp.bfloat16),
            pltpu.VMEM_SHARED((TILES, 2, NPAD), jnp.float32),
            pltpu.SemaphoreType.DMA,
            pltpu.SemaphoreType.DMA,
            pltpu.SemaphoreType.DMA,
        ],
    )


def _tc_body(xs, fxr, bxr, osc, W, Ws, bg, GM, gmb, FM, fmb, out_ref):
    pid = pl.program_id(0)
    x = xs[...].astype(jnp.float32)
    eaf = osc[0, 0, :]
    degf = osc[0, 1, :]
    eab = osc[1, 0, :]
    degb = osc[1, 1, :]
    Wv = W[...]
    S = jnp.stack([eaf, degf, eab, degb], axis=1)
    h = (jnp.dot(x * degf[:, None], Wv[0:32], preferred_element_type=jnp.float32)
         + jnp.dot(x * degb[:, None], Wv[32:64], preferred_element_type=jnp.float32)
         + jnp.dot(fxr[0].astype(jnp.float32), Wv[64:96], preferred_element_type=jnp.float32)
         + jnp.dot(bxr[0].astype(jnp.float32), Wv[96:128], preferred_element_type=jnp.float32)
         + jnp.dot(S, Ws[...], preferred_element_type=jnp.float32) + bg[...])
    g = jax.nn.sigmoid(jnp.dot(h, GM[...], preferred_element_type=jnp.float32)
                       + gmb[...])
    hv = jnp.dot(h, FM[...], preferred_element_type=jnp.float32) + fmb[...]
    rid = pid * BLK + lax.broadcasted_iota(jnp.int32, (BLK, 1), 0)
    part = jnp.sum(jnp.where(rid < N, g * hv, 0.0), axis=0, keepdims=True)

    @pl.when(pid == 0)
    def _():
        out_ref[...] = jnp.zeros_like(out_ref)

    out_ref[...] += part


def _tc_reduce(xs, oxx, osc, W, Ws, bg, GM, gmb, FM, fmb):
    return pl.pallas_call(
        _tc_body,
        grid=(NPAD // BLK,),
        in_specs=[
            pl.BlockSpec((BLK, 32), lambda i: (i, 0)),
            pl.BlockSpec((1, BLK, 32), lambda i: (0, i, 0)),
            pl.BlockSpec((1, BLK, 32), lambda i: (1, i, 0)),
            pl.BlockSpec((2, 2, BLK), lambda i: (0, 0, i)),
            pl.BlockSpec((128, 64), lambda i: (0, 0)),
            pl.BlockSpec((4, 64), lambda i: (0, 0)),
            pl.BlockSpec((1, 64), lambda i: (0, 0)),
            pl.BlockSpec((64, 64), lambda i: (0, 0)),
            pl.BlockSpec((1, 64), lambda i: (0, 0)),
            pl.BlockSpec((64, 64), lambda i: (0, 0)),
            pl.BlockSpec((1, 64), lambda i: (0, 0)),
        ],
        out_specs=pl.BlockSpec((1, 64), lambda i: (0, 0)),
        out_shape=jax.ShapeDtypeStruct((1, 64), jnp.float32),
    )(xs, oxx, oxx, osc, W, Ws, bg, GM, gmb, FM, fmb)


def _direction_mats(stack_params, nd_w, nd_b, Ag):
    A, b = _affine_of_stack(2, stack_params)
    Wnd = nd_w.reshape(P * B, HID + 1).T
    M = A @ Wnd[:HID]
    w_ea = Wnd[HID]
    cst = b @ Wnd[:HID] + nd_b.reshape(P * B)
    return M[:32] @ Ag, M[32:] @ Ag, w_ea @ Ag, cst @ Ag


def kernel(x, edge_attr, edge_index,
           fwd_c1_w, fwd_c1_b, fwd_c2_w, fwd_c2_b, fwd_c3_w, fwd_c3_b,
           fwd_lin_w, fwd_lin_b, fwd_nd_w, fwd_nd_b,
           bwd_c1_w, bwd_c1_b, bwd_c2_w, bwd_c2_b, bwd_c3_w, bwd_c3_b,
           bwd_lin_w, bwd_lin_b, bwd_nd_w, bwd_nd_b,
           g_c1_w, g_c1_b, g_c2_w, g_c2_b, g_c3_w, g_c3_b,
           g_lin_w, g_lin_b, gm_w, gm_b, fm_w, fm_b):
    f32 = jnp.float32
    Ag, bg = _affine_of_stack(1, (g_c1_w, g_c1_b, g_c2_w, g_c2_b,
                                  g_c3_w, g_c3_b, g_lin_w, g_lin_b))
    Mfi, Mfj, wf, cf = _direction_mats(
        (fwd_c1_w, fwd_c1_b, fwd_c2_w, fwd_c2_b, fwd_c3_w, fwd_c3_b,
         fwd_lin_w, fwd_lin_b), fwd_nd_w, fwd_nd_b, Ag)
    Mbi, Mbj, wb, cb = _direction_mats(
        (bwd_c1_w, bwd_c1_b, bwd_c2_w, bwd_c2_b, bwd_c3_w, bwd_c3_b,
         bwd_lin_w, bwd_lin_b), bwd_nd_w, bwd_nd_b, Ag)
    W_all = jnp.concatenate([Mfi, Mbi, Mfj, Mbj], axis=0)
    Ws = jnp.stack([wf, cf, wb, cb], axis=0)
    GM = jnp.zeros((HID, 64), f32).at[:, :GDIM].set(gm_w.T)
    gmb = jnp.zeros((1, 64), f32).at[0, :GDIM].set(gm_b)
    FM = jnp.zeros((HID, 64), f32).at[:, :GDIM].set(fm_w.T)
    fmb = jnp.zeros((1, 64), f32).at[0, :GDIM].set(fm_b)

    x32p = jnp.pad(x.reshape(N, P * B), ((0, NPAD - N), (0, 0))
                   ).astype(jnp.bfloat16)
    z32 = jnp.zeros((ROWS_PT, 32), jnp.bfloat16)
    z1 = jnp.zeros((NPAD,), f32)

    oxx, osc = _sc_segment()(x32p, edge_index, edge_attr, z32, z1)

    out = _tc_reduce(x32p, oxx, osc, W_all, Ws,
                     bg.reshape(1, HID), GM, gmb, FM, fmb)
    return out[0, :GDIM]
